# Initial kernel scaffold; baseline (speedup 1.0000x reference)
import jax
import jax.numpy as jnp
from jax.experimental import pallas as pl


def kernel(x, edge_index, batch, W0, b0, W1, b1):
    raise NotImplementedError("write your pallas kernel here")


# speedup vs baseline: 4.6120x; 4.6120x over previous
import functools

import jax
import jax.numpy as jnp
from jax import lax
from jax.experimental import pallas as pl
from jax.experimental.pallas import tpu as pltpu
from jax.experimental.pallas import tpu_sc as plsc

_N = 10000
_E = 320000
_D = 128
_B = 8

_NC = 2
_NS = 16
_NW = _NC * _NS
_EPW = _E // _NW
_C = 80
_NCHUNK = _EPW // _C
_RSTRIDE = 624
_RSIZE = 640

_BLK = 400
_GRID = _N // _BLK


def _make_mp_kernel():
    mesh = plsc.VectorSubcoreMesh(core_axis_name="c", subcore_axis_name="s")

    @functools.partial(
        pl.kernel,
        out_type=jax.ShapeDtypeStruct((_NC, _N, _D), jnp.float32),
        mesh=mesh,
        scratch_types=[
            pltpu.VMEM((_C,), jnp.int32),
            pltpu.VMEM((_C,), jnp.int32),
            pltpu.VMEM((_C, _D), jnp.float32),
            pltpu.VMEM_SHARED((_N, _D), jnp.float32),
            pltpu.SemaphoreType.DMA,
        ],
    )
    def mp(h_hbm, src_hbm, dst_hbm, zeros_hbm, out_hbm, src_v, dst_v,
           rows_v, acc, sem):
        c = lax.axis_index("c")
        s = lax.axis_index("s")
        pltpu.sync_copy(zeros_hbm, acc.at[pl.ds(s * _RSTRIDE, _RSIZE)])
        plsc.subcore_barrier()

        wid = s * _NC + c
        base = wid * _EPW

        def body(g, carry):
            off = base + g * _C
            pltpu.sync_copy(src_hbm.at[pl.ds(off, _C)], src_v)
            pltpu.sync_copy(dst_hbm.at[pl.ds(off, _C)], dst_v)
            pltpu.async_copy(h_hbm.at[src_v], rows_v, sem).wait()
            pltpu.sync_copy(rows_v, acc.at[dst_v], add=True)
            return carry

        lax.fori_loop(0, _NCHUNK, body, 0)
        plsc.subcore_barrier()
        pltpu.sync_copy(acc.at[pl.ds(s * _RSTRIDE, _RSIZE)],
                        out_hbm.at[c, pl.ds(s * _RSTRIDE, _RSIZE)])

    return mp


_mp_kernel = _make_mp_kernel()


def _mm_body(a0_ref, a1_ref, w_ref, b_ref, o_ref):
    agg = a0_ref[...] + a1_ref[...]
    h = jnp.dot(agg, w_ref[...], preferred_element_type=jnp.float32)
    o_ref[...] = jnp.maximum(h + b_ref[...], 0.0)


def _mm_relu(a0, a1, w, b):
    return pl.pallas_call(
        _mm_body,
        grid=(_GRID,),
        in_specs=[
            pl.BlockSpec((_BLK, _D), lambda i: (i, 0)),
            pl.BlockSpec((_BLK, _D), lambda i: (i, 0)),
            pl.BlockSpec((_D, _D), lambda i: (0, 0)),
            pl.BlockSpec((1, _D), lambda i: (0, 0)),
        ],
        out_specs=pl.BlockSpec((_BLK, _D), lambda i: (i, 0)),
        out_shape=jax.ShapeDtypeStruct((_N, _D), jnp.float32),
    )(a0, a1, w, b)


def _mm_pool_body(a0_ref, a1_ref, w_ref, b_ref, batch_ref, o_ref,
                  sums_ref, counts_ref):
    i = pl.program_id(0)

    @pl.when(i == 0)
    def _init():
        sums_ref[...] = jnp.zeros_like(sums_ref)
        counts_ref[...] = jnp.zeros_like(counts_ref)

    agg = a0_ref[...] + a1_ref[...]
    h = jnp.dot(agg, w_ref[...], preferred_element_type=jnp.float32)
    h = jnp.maximum(h + b_ref[...], 0.0)

    bvec = batch_ref[0, 0, :]
    onehot = (bvec[None, :] == lax.broadcasted_iota(jnp.int32, (_B, _BLK), 0)
              ).astype(jnp.float32)
    sums_ref[...] += jnp.dot(onehot, h, preferred_element_type=jnp.float32)
    counts_ref[...] += jnp.broadcast_to(
        jnp.sum(onehot, axis=1, keepdims=True), (_B, _D))

    @pl.when(i == _GRID - 1)
    def _fin():
        o_ref[...] = sums_ref[...] / jnp.maximum(counts_ref[...], 1.0)


def _mm_relu_pool(a0, a1, w, b, batch3d):
    return pl.pallas_call(
        _mm_pool_body,
        grid=(_GRID,),
        in_specs=[
            pl.BlockSpec((_BLK, _D), lambda i: (i, 0)),
            pl.BlockSpec((_BLK, _D), lambda i: (i, 0)),
            pl.BlockSpec((_D, _D), lambda i: (0, 0)),
            pl.BlockSpec((1, _D), lambda i: (0, 0)),
            pl.BlockSpec((1, 1, _BLK), lambda i: (i, 0, 0)),
        ],
        out_specs=pl.BlockSpec((_B, _D), lambda i: (0, 0)),
        out_shape=jax.ShapeDtypeStruct((_B, _D), jnp.float32),
        scratch_shapes=[
            pltpu.VMEM((_B, _D), jnp.float32),
            pltpu.VMEM((_B, _D), jnp.float32),
        ],
    )(a0, a1, w, b, batch3d)


@jax.jit
def kernel(x, edge_index, batch, W0, b0, W1, b1):
    src = edge_index[0]
    dst = edge_index[1]
    zeros = jnp.zeros((_RSIZE, _D), dtype=jnp.float32)

    parts = _mp_kernel(x, src, dst, zeros)
    h1 = _mm_relu(parts[0], parts[1], W0, b0.reshape(1, _D))

    parts2 = _mp_kernel(h1, src, dst, zeros)
    batch3d = batch.reshape(_GRID, 1, _BLK)
    emb = _mm_relu_pool(parts2[0], parts2[1], W1, b1.reshape(1, _D), batch3d)
    return emb
